# baseline (speedup 1.0000x reference)
import functools

import jax
import jax.numpy as jnp
from jax import lax
from jax.experimental import pallas as pl
from jax.experimental.pallas import tpu as pltpu
from jax.experimental.pallas import tpu_sc as plsc

V = 30000
D = 256
H = 256
B = 16
U = 15
W = 30
NROW = B * (U + 1)
P = NROW * W
NW = 32
NPW = NROW // NW
TCH = 6
WSPLIT = 18


def _make_sc(w0, nsteps, chunks):
    rpw = nsteps * NPW
    mesh = plsc.VectorSubcoreMesh(core_axis_name="c", subcore_axis_name="s")

    @functools.partial(
        pl.kernel,
        out_type=jax.ShapeDtypeStruct((nsteps, NROW, D), jnp.float32),
        mesh=mesh,
        compiler_params=pltpu.CompilerParams(needs_layout_passes=False),
        scratch_types=[
            pltpu.VMEM((P,), jnp.int32),
            pltpu.VMEM((128,), jnp.int32),
            pltpu.VMEM((rpw,), jnp.int32),
            pltpu.VMEM((rpw, D), jnp.float32),
            pltpu.SemaphoreType.DMA,
            pltpu.SemaphoreType.DMA,
        ],
    )
    def sc_kernel(toks_hbm, len_hbm, emb_hbm, x_hbm,
                  toks_v, len_v, idx_v, rows_v, gsem, osem):
        wid = lax.axis_index("s") * 2 + lax.axis_index("c")
        n0 = wid * NPW
        pltpu.sync_copy(toks_hbm, toks_v)
        pltpu.sync_copy(len_hbm, len_v.at[pl.ds(0, B)])
        lane = lax.iota(jnp.int32, 16)
        us = n0 >> 4
        b = (n0 & 15) + (lane & 7)
        lb = plsc.load_gather(len_v, [b])
        su = jnp.maximum(jnp.where(us < lb, us, us - 1), 0)
        off0 = jnp.where(lb == us,
                         B * U * W + b * W,
                         b * (U * W) + su * W)
        wbase = lane >> 3
        for i in range(rpw // 16):
            w = w0 + 2 * i + wbase
            idx_v[pl.ds(i * 16, 16)] = plsc.load_gather(toks_v, [off0 + w])
        off = 0
        gds = []
        for sz in chunks:
            gds.append(pltpu.async_copy(
                emb_hbm.at[idx_v.at[pl.ds(off, sz)]],
                rows_v.at[pl.ds(off, sz)], gsem))
            off += sz
        out_descs = []
        for ci, sz in enumerate(chunks):
            gds[ci].wait()
            base = sum(chunks[:ci])
            for wl in range(base // NPW, (base + sz) // NPW):
                out_descs.append(pltpu.async_copy(
                    rows_v.at[pl.ds(wl * NPW, NPW)],
                    x_hbm.at[wl, pl.ds(n0, NPW)], osem))
        for d in out_descs:
            d.wait()

    return sc_kernel


def _lens_from_raw(cul_ref, ql_ref):
    eye = (lax.broadcasted_iota(jnp.int32, (B, B), 0)
           == lax.broadcasted_iota(jnp.int32, (B, B), 1)).astype(jnp.float32)
    qlc = jnp.dot(eye * ql_ref[...].astype(jnp.float32).reshape(1, B),
                  jnp.ones((B, 1), jnp.float32),
                  preferred_element_type=jnp.float32)
    cc = jnp.concatenate([cul_ref[...].astype(jnp.float32), qlc], axis=1)
    rows = lax.broadcasted_iota(jnp.int32, (NROW, B), 0)
    cols = lax.broadcasted_iota(jnp.int32, (NROW, B), 1)
    sb = ((rows & 15) == cols).astype(jnp.float32)
    mu = ((rows >> 4) == cols).astype(jnp.float32)
    return jnp.dot(jnp.dot(sb, cc, preferred_element_type=jnp.float32) * mu,
                   jnp.ones((B, 1), jnp.float32),
                   preferred_element_type=jnp.float32)


def _sg(a):
    return 0.5 + 0.5 * jnp.tanh(0.5 * a)


def _gru_steps(h, gxc, whu_bf, lenv, t0):
    for tl in range(gxc.shape[0] // NROW):
        gx = gxc[tl * NROW:(tl + 1) * NROW, :]
        gh = jnp.dot(h.astype(jnp.bfloat16), whu_bf,
                     preferred_element_type=jnp.float32)
        r = _sg(gx[:, :H] + gh[:, :H])
        z = _sg(gx[:, H:2 * H] + gh[:, H:2 * H])
        nn = jnp.tanh(gx[:, 2 * H:] + r * gh[:, 2 * H:])
        h = jnp.where(lenv > t0 + tl, nn + z * (h - nn), h)
    return h


def _tc_a(x_tm, cul, ql, wx_u, wh_u, b_u):
    nch = WSPLIT // TCH

    def body(x_ref, cul_ref, ql_ref, wxu_ref, whu_ref, bu_ref, out_ref,
             h_ref, len_ref):
        i = pl.program_id(0)

        @pl.when(i == 0)
        def _():
            h_ref[...] = jnp.zeros((NROW, H), jnp.float32)
            len_ref[...] = _lens_from_raw(cul_ref, ql_ref)

        gxc = (jnp.dot(x_ref[...].reshape(TCH * NROW, D).astype(jnp.bfloat16),
                       wxu_ref[...].astype(jnp.bfloat16),
                       preferred_element_type=jnp.float32) + bu_ref[...])
        h = _gru_steps(h_ref[...], gxc, whu_ref[...].astype(jnp.bfloat16),
                       len_ref[...], i * TCH)
        h_ref[...] = h

        @pl.when(i == nch - 1)
        def _():
            out_ref[...] = h

    full = lambda shape: pl.BlockSpec(shape, lambda i: tuple(0 for _ in shape))
    return pl.pallas_call(
        body,
        grid=(nch,),
        in_specs=[
            pl.BlockSpec((TCH, NROW, D), lambda i: (i, 0, 0)),
            full((B, U)), full((B,)),
            full((D, 3 * H)), full((H, 3 * H)), full((3 * H,)),
        ],
        out_specs=full((NROW, H)),
        out_shape=jax.ShapeDtypeStruct((NROW, H), jnp.float32),
        scratch_shapes=[pltpu.VMEM((NROW, H), jnp.float32),
                        pltpu.VMEM((NROW, 1), jnp.float32)],
    )(x_tm, cul, ql, wx_u, wh_u, b_u)


def _tc_b(h_in, x_tm, cul, ql, ctx_len, wx_u, wh_u, b_u, wx_c, wh_c, b_c):
    nch = (W - WSPLIT) // TCH

    def body(h_in_ref, x_ref, cul_ref, ql_ref, cl_ref, wxu_ref, whu_ref,
             bu_ref, wxc_ref, whc_ref, bc_ref, out_ref, h_ref, len_ref):
        i = pl.program_id(0)

        @pl.when(i == 0)
        def _():
            h_ref[...] = h_in_ref[...]
            len_ref[...] = _lens_from_raw(cul_ref, ql_ref)

        gxc = (jnp.dot(x_ref[...].reshape(TCH * NROW, D).astype(jnp.bfloat16),
                       wxu_ref[...].astype(jnp.bfloat16),
                       preferred_element_type=jnp.float32) + bu_ref[...])
        h = _gru_steps(h_ref[...], gxc, whu_ref[...].astype(jnp.bfloat16),
                       len_ref[...], WSPLIT + i * TCH)
        h_ref[...] = h

        @pl.when(i == nch - 1)
        def _():
            g2 = (jnp.dot(h.astype(jnp.bfloat16),
                          wxc_ref[...].astype(jnp.bfloat16),
                          preferred_element_type=jnp.float32) + bc_ref[...])
            whc = whc_ref[...].astype(jnp.bfloat16)
            eye = (lax.broadcasted_iota(jnp.int32, (B, B), 0)
                   == lax.broadcasted_iota(jnp.int32, (B, B), 1)
                   ).astype(jnp.float32)
            clv = jnp.dot(eye * cl_ref[...].astype(jnp.float32).reshape(1, B),
                          jnp.ones((B, 1), jnp.float32),
                          preferred_element_type=jnp.float32)
            h2 = jnp.zeros((B, H), jnp.float32)
            for s in range(U + 1):
                gx2 = g2[s * B:(s + 1) * B, :]
                gh2 = jnp.dot(h2.astype(jnp.bfloat16), whc,
                              preferred_element_type=jnp.float32)
                r2 = _sg(gx2[:, :H] + gh2[:, :H])
                z2 = _sg(gx2[:, H:2 * H] + gh2[:, H:2 * H])
                n2 = jnp.tanh(gx2[:, 2 * H:] + r2 * gh2[:, 2 * H:])
                h2 = jnp.where(clv >= s, n2 + z2 * (h2 - n2), h2)
            out_ref[...] = h2

    full = lambda shape: pl.BlockSpec(shape, lambda i: tuple(0 for _ in shape))
    return pl.pallas_call(
        body,
        grid=(nch,),
        in_specs=[
            full((NROW, H)),
            pl.BlockSpec((TCH, NROW, D), lambda i: (i, 0, 0)),
            full((B, U)), full((B,)), full((B,)),
            full((D, 3 * H)), full((H, 3 * H)), full((3 * H,)),
            full((H, 3 * H)), full((H, 3 * H)), full((3 * H,)),
        ],
        out_specs=full((B, H)),
        out_shape=jax.ShapeDtypeStruct((B, H), jnp.float32),
        scratch_shapes=[pltpu.VMEM((NROW, H), jnp.float32),
                        pltpu.VMEM((NROW, 1), jnp.float32)],
    )(h_in, x_tm, cul, ql, ctx_len, wx_u, wh_u, b_u, wx_c, wh_c, b_c)


def kernel(contexts, context_utterance_lengths, context_lengths, queries,
           query_lengths, emb, Wx_u, Wh_u, b_u, Wx_c, Wh_c, b_c):
    toks = jnp.concatenate([contexts.reshape(-1), queries.reshape(-1)])
    sc_a = _make_sc(0, WSPLIT, (72, 72))
    sc_b = _make_sc(WSPLIT, W - WSPLIT, (96,))
    x_a = sc_a(toks, context_lengths, emb)
    x_b = sc_b(toks, context_lengths, emb)
    h_mid = _tc_a(x_a, context_utterance_lengths, query_lengths,
                  Wx_u, Wh_u, b_u)
    return _tc_b(h_mid, x_b, context_utterance_lengths, query_lengths,
                 context_lengths, Wx_u, Wh_u, b_u, Wx_c, Wh_c, b_c)


# speedup vs baseline: 1.0212x; 1.0212x over previous
import functools

import jax
import jax.numpy as jnp
from jax import lax
from jax.experimental import pallas as pl
from jax.experimental.pallas import tpu as pltpu
from jax.experimental.pallas import tpu_sc as plsc

V = 30000
D = 256
H = 256
B = 16
U = 15
W = 30
NROW = B * (U + 1)
P = NROW * W
NW = 32
RPW = P // NW
NPW = NROW // NW
CHUNK = 80
WPC = CHUNK // NPW
TCH = 10
NCH = W // TCH


def _sc_gather(toks, ctx_len, emb):
    mesh = plsc.VectorSubcoreMesh(core_axis_name="c", subcore_axis_name="s")

    @functools.partial(
        pl.kernel,
        out_type=jax.ShapeDtypeStruct((W, NROW, D), jnp.float32),
        mesh=mesh,
        compiler_params=pltpu.CompilerParams(needs_layout_passes=False),
        scratch_types=[
            pltpu.VMEM((P,), jnp.int32),
            pltpu.VMEM((128,), jnp.int32),
            pltpu.VMEM((RPW,), jnp.int32),
            pltpu.VMEM((RPW, D), jnp.float32),
            pltpu.SemaphoreType.DMA,
            pltpu.SemaphoreType.DMA,
        ],
    )
    def sc_kernel(toks_hbm, len_hbm, emb_hbm, x_hbm,
                  toks_v, len_v, idx_v, rows_v, gsem, osem):
        wid = lax.axis_index("s") * 2 + lax.axis_index("c")
        n0 = wid * NPW
        pltpu.sync_copy(toks_hbm, toks_v)
        pltpu.sync_copy(len_hbm, len_v.at[pl.ds(0, B)])
        lane = lax.iota(jnp.int32, 16)
        us = n0 >> 4
        b = (n0 & 15) + (lane & 7)
        lb = plsc.load_gather(len_v, [b])
        su = jnp.maximum(jnp.where(us < lb, us, us - 1), 0)
        off0 = jnp.where(lb == us,
                         B * U * W + b * W,
                         b * (U * W) + su * W)
        wbase = lane >> 3
        out_descs = []
        for c in range(RPW // CHUNK):
            for i in range(CHUNK // 16):
                w = c * WPC + 2 * i + wbase
                idx_v[pl.ds(c * CHUNK + i * 16, 16)] = plsc.load_gather(
                    toks_v, [off0 + w])
            gd = pltpu.async_copy(
                emb_hbm.at[idx_v.at[pl.ds(c * CHUNK, CHUNK)]],
                rows_v.at[pl.ds(c * CHUNK, CHUNK)],
                gsem,
            )
            if c > 0:
                for w in range((c - 1) * WPC, c * WPC):
                    out_descs.append(pltpu.async_copy(
                        rows_v.at[pl.ds(w * NPW, NPW)],
                        x_hbm.at[w, pl.ds(n0, NPW)],
                        osem,
                    ))
            gd.wait()
        for w in range((RPW // CHUNK - 1) * WPC, W):
            out_descs.append(pltpu.async_copy(
                rows_v.at[pl.ds(w * NPW, NPW)],
                x_hbm.at[w, pl.ds(n0, NPW)],
                osem,
            ))
        for d in out_descs:
            d.wait()

    return sc_kernel(toks, ctx_len, emb)


def _tc_gru(x_tm, cul, ql, ctx_len, wx_u, wh_u, b_u, wx_c, wh_c, b_c):
    def sg(a):
        return 0.5 + 0.5 * jnp.tanh(0.5 * a)

    def col16(row):
        eye = (lax.broadcasted_iota(jnp.int32, (B, B), 0)
               == lax.broadcasted_iota(jnp.int32, (B, B), 1)).astype(jnp.float32)
        return jnp.dot(eye * row, jnp.ones((B, 1), jnp.float32),
                       preferred_element_type=jnp.float32)

    def tc_kernel(x_ref, cul_ref, ql_ref, cl_ref, wxu_ref, whu_ref, bu_ref,
                  wxc_ref, whc_ref, bc_ref, out_ref, h_ref, len_ref):
        i = pl.program_id(0)

        @pl.when(i == 0)
        def _():
            h_ref[...] = jnp.zeros((NROW, H), jnp.float32)
            qlc = col16(ql_ref[...].astype(jnp.float32).reshape(1, B))
            cc = jnp.concatenate(
                [cul_ref[...].astype(jnp.float32), qlc], axis=1)
            rows = lax.broadcasted_iota(jnp.int32, (NROW, B), 0)
            cols = lax.broadcasted_iota(jnp.int32, (NROW, B), 1)
            sb = ((rows & 15) == cols).astype(jnp.float32)
            mu = ((rows >> 4) == cols).astype(jnp.float32)
            len_ref[...] = jnp.dot(
                jnp.dot(sb, cc, preferred_element_type=jnp.float32) * mu,
                jnp.ones((B, 1), jnp.float32),
                preferred_element_type=jnp.float32)

        whu = whu_ref[...].astype(jnp.bfloat16)
        lenv = len_ref[...]
        gxc = (jnp.dot(x_ref[...].reshape(TCH * NROW, D).astype(jnp.bfloat16),
                       wxu_ref[...].astype(jnp.bfloat16),
                       preferred_element_type=jnp.float32) + bu_ref[...])
        h = h_ref[...]
        for tl in range(TCH):
            gx = gxc[tl * NROW:(tl + 1) * NROW, :]
            gh = jnp.dot(h.astype(jnp.bfloat16), whu,
                         preferred_element_type=jnp.float32)
            r = sg(gx[:, :H] + gh[:, :H])
            z = sg(gx[:, H:2 * H] + gh[:, H:2 * H])
            nn = jnp.tanh(gx[:, 2 * H:] + r * gh[:, 2 * H:])
            t = i * TCH + tl
            h = jnp.where(lenv > t, nn + z * (h - nn), h)
        h_ref[...] = h

        @pl.when(i == NCH - 1)
        def _():
            g2 = (jnp.dot(h.astype(jnp.bfloat16),
                          wxc_ref[...].astype(jnp.bfloat16),
                          preferred_element_type=jnp.float32) + bc_ref[...])
            whc = whc_ref[...].astype(jnp.bfloat16)
            clv = col16(cl_ref[...].astype(jnp.float32).reshape(1, B))
            h2 = jnp.zeros((B, H), jnp.float32)
            for s in range(U + 1):
                gx2 = g2[s * B:(s + 1) * B, :]
                gh2 = jnp.dot(h2.astype(jnp.bfloat16), whc,
                          preferred_element_type=jnp.float32)
                r2 = sg(gx2[:, :H] + gh2[:, :H])
                z2 = sg(gx2[:, H:2 * H] + gh2[:, H:2 * H])
                n2 = jnp.tanh(gx2[:, 2 * H:] + r2 * gh2[:, 2 * H:])
                h2 = jnp.where(clv >= s, n2 + z2 * (h2 - n2), h2)
            out_ref[...] = h2

    full = lambda shape: pl.BlockSpec(shape, lambda i: tuple(0 for _ in shape))
    return pl.pallas_call(
        tc_kernel,
        grid=(NCH,),
        in_specs=[
            pl.BlockSpec((TCH, NROW, D), lambda i: (i, 0, 0)),
            full((B, U)),
            full((B,)),
            full((B,)),
            full((D, 3 * H)),
            full((H, 3 * H)),
            full((3 * H,)),
            full((H, 3 * H)),
            full((H, 3 * H)),
            full((3 * H,)),
        ],
        out_specs=full((B, H)),
        out_shape=jax.ShapeDtypeStruct((B, H), jnp.float32),
        scratch_shapes=[pltpu.VMEM((NROW, H), jnp.float32),
                        pltpu.VMEM((NROW, 1), jnp.float32)],
    )(x_tm, cul, ql, ctx_len, wx_u, wh_u, b_u, wx_c, wh_c, b_c)


def kernel(contexts, context_utterance_lengths, context_lengths, queries,
           query_lengths, emb, Wx_u, Wh_u, b_u, Wx_c, Wh_c, b_c):
    toks = jnp.concatenate([contexts.reshape(-1), queries.reshape(-1)])
    x_tm = _sc_gather(toks, context_lengths, emb)
    return _tc_gru(x_tm, context_utterance_lengths, query_lengths,
                   context_lengths, Wx_u, Wh_u, b_u, Wx_c, Wh_c, b_c)
